# baseline (speedup 1.0000x reference)
import functools

import jax
import jax.numpy as jnp
from jax import lax
from jax.experimental import pallas as pl
from jax.experimental.pallas import tpu as pltpu
from jax.experimental.pallas import tpu_sc as plsc

EMBED_DIM = 64
CHUNK = 128


@functools.partial(jax.jit, static_argnames=("total",))
def _flat_gather(idx_flat, table, total):
    info = plsc.get_sparse_core_info()
    num_workers = info.num_cores * info.num_subcores
    per_worker = total // num_workers
    n_chunks = per_worker // CHUNK
    mesh = plsc.VectorSubcoreMesh(core_axis_name="c", subcore_axis_name="s")

    @functools.partial(
        pl.kernel,
        mesh=mesh,
        compiler_params=pltpu.CompilerParams(use_tc_tiling_on_sc=False),
        out_type=jax.ShapeDtypeStruct((total, EMBED_DIM), jnp.float32),
        scratch_types=[
            pltpu.VMEM((per_worker,), jnp.int32),
            pltpu.VMEM((CHUNK, EMBED_DIM), jnp.float32),
            pltpu.SemaphoreType.DMA,
        ],
    )
    def k(idx_hbm, table_hbm, out_hbm, idx_v, rows_v, sem):
        wid = lax.axis_index("s") * info.num_cores + lax.axis_index("c")
        base = wid * per_worker
        pltpu.sync_copy(idx_hbm.at[pl.ds(base, per_worker)], idx_v)

        def body(g, carry):
            off = pl.multiple_of(g * CHUNK, CHUNK)
            pltpu.async_copy(
                table_hbm.at[idx_v.at[pl.ds(off, CHUNK)]], rows_v, sem
            ).wait()
            pltpu.sync_copy(rows_v, out_hbm.at[pl.ds(base + off, CHUNK)])
            return carry

        lax.fori_loop(0, n_chunks, body, 0)

    return k(idx_flat, table)


def kernel(seqTensor, table):
    batch, hist = seqTensor.shape
    total = batch * hist
    idx_flat = seqTensor.reshape(total).astype(jnp.int32)
    out = _flat_gather(idx_flat, table, total)
    return out.reshape(batch, hist, EMBED_DIM)


# speedup vs baseline: 1.1136x; 1.1136x over previous
import functools

import jax
import jax.numpy as jnp
from jax import lax
from jax.experimental import pallas as pl
from jax.experimental.pallas import tpu as pltpu
from jax.experimental.pallas import tpu_sc as plsc

EMBED_DIM = 64
CHUNK = 128
K = 4
GROUP = K * CHUNK


@functools.partial(jax.jit, static_argnames=("total",))
def _flat_gather(idx_flat, table, total):
    info = plsc.get_sparse_core_info()
    num_workers = info.num_cores * info.num_subcores
    per_worker = total // num_workers
    n_groups = per_worker // GROUP
    mesh = plsc.VectorSubcoreMesh(core_axis_name="c", subcore_axis_name="s")

    @functools.partial(
        pl.kernel,
        mesh=mesh,
        compiler_params=pltpu.CompilerParams(use_tc_tiling_on_sc=False),
        out_type=jax.ShapeDtypeStruct((total, EMBED_DIM), jnp.float32),
        scratch_types=[
            pltpu.VMEM((per_worker,), jnp.int32),
            pltpu.VMEM((GROUP, EMBED_DIM), jnp.float32),
            pltpu.VMEM((GROUP, EMBED_DIM), jnp.float32),
            pltpu.SemaphoreType.DMA,
            pltpu.SemaphoreType.DMA,
            pltpu.SemaphoreType.DMA,
            pltpu.SemaphoreType.DMA,
        ],
    )
    def k(idx_hbm, table_hbm, out_hbm, idx_v, rows_a, rows_b,
          gsem_a, gsem_b, wsem_a, wsem_b):
        wid = lax.axis_index("s") * info.num_cores + lax.axis_index("c")
        base = wid * per_worker
        pltpu.sync_copy(idx_hbm.at[pl.ds(base, per_worker)], idx_v)

        rows = (rows_a, rows_b)
        gsem = (gsem_a, gsem_b)
        wsem = (wsem_a, wsem_b)

        def issue_gathers(g, x):
            for j in range(K):
                off = pl.multiple_of(g * GROUP + j * CHUNK, CHUNK)
                pltpu.async_copy(
                    table_hbm.at[idx_v.at[pl.ds(off, CHUNK)]],
                    rows[x].at[pl.ds(j * CHUNK, CHUNK)],
                    gsem[x],
                )

        def drain_gathers(x):
            pltpu.make_async_copy(
                table_hbm.at[pl.ds(0, GROUP)], rows[x], gsem[x]
            ).wait()

        def drain_write(x):
            pltpu.make_async_copy(
                rows[x], out_hbm.at[pl.ds(base, GROUP)], wsem[x]
            ).wait()

        issue_gathers(0, 0)

        def body(grp, x):
            y = 1 - x

            @pl.when(grp + 1 < n_groups)
            def _():
                @pl.when(grp >= 1)
                def _():
                    drain_write(y)
                issue_gathers(grp + 1, y)

            drain_gathers(x)
            woff = pl.multiple_of(base + grp * GROUP, GROUP)
            pltpu.async_copy(rows[x], out_hbm.at[pl.ds(woff, GROUP)], wsem[x])

        def pair_body(p, carry):
            body(2 * p, 0)
            body(2 * p + 1, 1)
            return carry

        lax.fori_loop(0, n_groups // 2, pair_body, 0)
        drain_write(0)
        drain_write(1)

    return k(idx_flat, table)


def kernel(seqTensor, table):
    batch, hist = seqTensor.shape
    total = batch * hist
    idx_flat = seqTensor.reshape(total).astype(jnp.int32)
    out = _flat_gather(idx_flat, table, total)
    return out.reshape(batch, hist, EMBED_DIM)
